# baseline (speedup 1.0000x reference)
import functools

import jax
import jax.numpy as jnp
from jax import lax
from jax.experimental import pallas as pl
from jax.experimental.pallas import tpu as pltpu
from jax.experimental.pallas import tpu_sc as plsc

BATCH = 16384
DIM = 128
L = 16
NC = 2
NS = 16
NW = NC * NS
BPW = BATCH // NW
GCH = 128
NG = BPW // GCH
NCH = 4
RPC = BPW // NCH
GPC = RPC // L


def _body(factor_hbm, feat_hbm, w_hbm, out_hbm,
          fac0, fac1, fac2, fac3, idx_v, wg_v, part_v,
          sem_g, sem0, sem1, sem2, sem3):
    c = lax.axis_index("c")
    s = lax.axis_index("s")
    wid = s * NC + c
    base = wid * BPW

    bufs = (fac0, fac1, fac2, fac3)
    sems = (sem0, sem1, sem2, sem3)
    cps = [
        pltpu.async_copy(
            factor_hbm.at[pl.ds((base + ci * RPC) * DIM, RPC * DIM)],
            bufs[ci], sems[ci])
        for ci in range(NCH)
    ]
    pltpu.sync_copy(feat_hbm.at[wid], idx_v)
    gathers = [
        pltpu.async_copy(w_hbm.at[idx_v.at[j]], wg_v.at[pl.ds(j * GCH, GCH)], sem_g)
        for j in range(NG)
    ]
    for g in gathers:
        g.wait()

    acc = jnp.zeros((L,), jnp.float32)
    for ci in range(NCH):
        cps[ci].wait()
        fac = bufs[ci]

        def group_step(g, a, _ci=ci, _fac=fac):
            w16 = wg_v[pl.ds((_ci * GPC + g) * L, L)]
            gbase = g * L * DIM
            for k in range(L):
                w_vec = jnp.full((L,), w16[k], jnp.float32)
                rbase = gbase + k * DIM
                sq = None
                for ch in range(DIM // L):
                    v = _fac[pl.ds(rbase + ch * L, L)]
                    vv = v * v
                    sq = vv if sq is None else sq + vv
                a = a + w_vec * sq
            return a

        acc = lax.fori_loop(0, GPC, group_step, acc)

    part_v[...] = acc
    pltpu.sync_copy(part_v, out_hbm.at[wid])


@jax.jit
def _sc_call(factor_flat, feat3d, weights_flat):
    mesh = plsc.VectorSubcoreMesh(core_axis_name="c", subcore_axis_name="s")
    kern = functools.partial(
        pl.kernel,
        mesh=mesh,
        out_type=jax.ShapeDtypeStruct((NW, L), jnp.float32),
        scratch_types=[
            pltpu.VMEM((RPC * DIM,), jnp.float32),
            pltpu.VMEM((RPC * DIM,), jnp.float32),
            pltpu.VMEM((RPC * DIM,), jnp.float32),
            pltpu.VMEM((RPC * DIM,), jnp.float32),
            pltpu.VMEM((NG, GCH), jnp.int32),
            pltpu.VMEM((BPW,), jnp.float32),
            pltpu.VMEM((L,), jnp.float32),
            pltpu.SemaphoreType.DMA,
            pltpu.SemaphoreType.DMA,
            pltpu.SemaphoreType.DMA,
            pltpu.SemaphoreType.DMA,
            pltpu.SemaphoreType.DMA,
        ],
    )(_body)
    return kern(factor_flat, feat3d, weights_flat)


def kernel(factor, features, weights):
    factor_flat = factor.reshape(-1)
    feat3d = features.astype(jnp.int32).reshape(NW, NG, GCH)
    weights_flat = weights.reshape(-1)
    parts = _sc_call(factor_flat, feat3d, weights_flat)
    return jnp.sum(parts)


# speedup vs baseline: 1.0112x; 1.0112x over previous
import functools

import jax
import jax.numpy as jnp
from jax import lax
from jax.experimental import pallas as pl
from jax.experimental.pallas import tpu as pltpu
from jax.experimental.pallas import tpu_sc as plsc

BATCH = 16384
DIM = 128
L = 16
NC = 2
NS = 16
NW = NC * NS
BPW = BATCH // NW
GCH = 128
NG = BPW // GCH
NCH = 4
RPC = BPW // NCH
GPC = RPC // L


def _body(factor_hbm, feat_hbm, w_hbm, out_hbm,
          fac0, fac1, fac2, fac3, idx_v, wg_v, part_v,
          sem_g, sem0, sem1, sem2, sem3):
    c = lax.axis_index("c")
    s = lax.axis_index("s")
    wid = s * NC + c
    base = wid * BPW

    bufs = (fac0, fac1, fac2, fac3)
    sems = (sem0, sem1, sem2, sem3)
    pltpu.sync_copy(feat_hbm.at[wid], idx_v)
    gathers = [
        pltpu.async_copy(w_hbm.at[idx_v.at[j]], wg_v.at[pl.ds(j * GCH, GCH)], sem_g)
        for j in range(NG)
    ]
    cps = [
        pltpu.async_copy(
            factor_hbm.at[pl.ds((base + ci * RPC) * DIM, RPC * DIM)],
            bufs[ci], sems[ci])
        for ci in range(NCH)
    ]
    for g in gathers:
        g.wait()

    acc = jnp.zeros((L,), jnp.float32)
    for ci in range(NCH):
        cps[ci].wait()
        fac = bufs[ci]

        def group_step(g, a, _ci=ci, _fac=fac):
            w16 = wg_v[pl.ds((_ci * GPC + g) * L, L)]
            gbase = g * L * DIM
            for k in range(L):
                w_vec = jnp.full((L,), w16[k], jnp.float32)
                rbase = gbase + k * DIM
                sq = None
                for ch in range(DIM // L):
                    v = _fac[pl.ds(rbase + ch * L, L)]
                    vv = v * v
                    sq = vv if sq is None else sq + vv
                a = a + w_vec * sq
            return a

        acc = lax.fori_loop(0, GPC, group_step, acc)

    part_v[...] = acc
    pltpu.sync_copy(part_v, out_hbm.at[wid])


@jax.jit
def _sc_call(factor_flat, feat3d, weights_flat):
    mesh = plsc.VectorSubcoreMesh(core_axis_name="c", subcore_axis_name="s")
    kern = functools.partial(
        pl.kernel,
        mesh=mesh,
        out_type=jax.ShapeDtypeStruct((NW, L), jnp.float32),
        scratch_types=[
            pltpu.VMEM((RPC * DIM,), jnp.float32),
            pltpu.VMEM((RPC * DIM,), jnp.float32),
            pltpu.VMEM((RPC * DIM,), jnp.float32),
            pltpu.VMEM((RPC * DIM,), jnp.float32),
            pltpu.VMEM((NG, GCH), jnp.int32),
            pltpu.VMEM((BPW,), jnp.float32),
            pltpu.VMEM((L,), jnp.float32),
            pltpu.SemaphoreType.DMA,
            pltpu.SemaphoreType.DMA,
            pltpu.SemaphoreType.DMA,
            pltpu.SemaphoreType.DMA,
            pltpu.SemaphoreType.DMA,
        ],
    )(_body)
    return kern(factor_flat, feat3d, weights_flat)


def kernel(factor, features, weights):
    factor_flat = factor.reshape(-1)
    feat3d = features.astype(jnp.int32).reshape(NW, NG, GCH)
    weights_flat = weights.reshape(-1)
    parts = _sc_call(factor_flat, feat3d, weights_flat)
    return jnp.sum(parts)


# speedup vs baseline: 1.1508x; 1.1381x over previous
import functools

import jax
import jax.numpy as jnp
from jax import lax
from jax.experimental import pallas as pl
from jax.experimental.pallas import tpu as pltpu
from jax.experimental.pallas import tpu_sc as plsc

BATCH = 16384
DIM = 128
L = 16
NC = 2
NS = 16
NW = NC * NS
BPW = BATCH // NW
GCH = 128
NG = BPW // GCH
NCH = 4
RPC = BPW // NCH
GPC = RPC // L
NGRP = BPW // L


def _body(factor_hbm, feat_hbm, w_hbm, out_hbm,
          fac_v, idx_v, wg_v, part_v,
          sem_g, sem0, sem1, sem2, sem3):
    c = lax.axis_index("c")
    s = lax.axis_index("s")
    wid = s * NC + c
    base = wid * BPW

    pltpu.sync_copy(feat_hbm.at[wid], idx_v)
    gathers = [
        pltpu.async_copy(w_hbm.at[idx_v.at[j]], wg_v.at[pl.ds(j * GCH, GCH)], sem_g)
        for j in range(NG)
    ]
    sems = (sem0, sem1, sem2, sem3)
    cps = [
        pltpu.async_copy(
            factor_hbm.at[pl.ds((base + ci * RPC) * DIM, RPC * DIM)],
            fac_v.at[pl.ds(ci * RPC * DIM, RPC * DIM)], sems[ci])
        for ci in range(NCH)
    ]
    for g in gathers:
        g.wait()
    cps[0].wait()

    def group_step(g, a):
        for ci in range(1, NCH):
            @pl.when(g == ci * GPC)
            def _wait(_ci=ci):
                cps[_ci].wait()
        w16 = wg_v[pl.ds(g * L, L)]
        gbase = g * L * DIM
        for k in range(L):
            w_vec = jnp.full((L,), w16[k], jnp.float32)
            rbase = gbase + k * DIM
            sq = None
            for ch in range(DIM // L):
                v = fac_v[pl.ds(rbase + ch * L, L)]
                vv = v * v
                sq = vv if sq is None else sq + vv
            a = a + w_vec * sq
        return a

    acc = lax.fori_loop(0, NGRP, group_step, jnp.zeros((L,), jnp.float32))
    part_v[...] = acc
    pltpu.sync_copy(part_v, out_hbm.at[wid])


@jax.jit
def _sc_call(factor_flat, feat3d, weights_flat):
    mesh = plsc.VectorSubcoreMesh(core_axis_name="c", subcore_axis_name="s")
    kern = functools.partial(
        pl.kernel,
        mesh=mesh,
        out_type=jax.ShapeDtypeStruct((NW, L), jnp.float32),
        scratch_types=[
            pltpu.VMEM((BPW * DIM,), jnp.float32),
            pltpu.VMEM((NG, GCH), jnp.int32),
            pltpu.VMEM((BPW,), jnp.float32),
            pltpu.VMEM((L,), jnp.float32),
            pltpu.SemaphoreType.DMA,
            pltpu.SemaphoreType.DMA,
            pltpu.SemaphoreType.DMA,
            pltpu.SemaphoreType.DMA,
            pltpu.SemaphoreType.DMA,
        ],
    )(_body)
    return kern(factor_flat, feat3d, weights_flat)


def kernel(factor, features, weights):
    factor_flat = factor.reshape(-1)
    feat3d = features.astype(jnp.int32).reshape(NW, NG, GCH)
    weights_flat = weights.reshape(-1)
    parts = _sc_call(factor_flat, feat3d, weights_flat)
    return jnp.sum(parts)
